# speedup vs baseline: 7.1541x; 7.1541x over previous
import functools

import jax
import jax.numpy as jnp
import numpy as np
from jax import lax
from jax.experimental import pallas as pl
from jax.experimental.pallas import tpu as pltpu

NUM_EMB = [100000] * 26
EMBED_DIM = 32
BATCH = 4096
N_FIELDS = 26
N_CONCAT = N_FIELDS + 1
BB = 512


def _dense_body(dense_ref, sparse_ref, wb0_ref, bb0_ref, wb1_ref, bb1_ref,
                wb2_ref, bb2_ref, ahat_ref, wt0d_ref, bt0_ref, wt1_ref,
                bt1_ref, wt2_ref, bt2_ref, out_ref, xs_ref):
    xd = dense_ref[...]
    h = jnp.maximum(xd @ wb0_ref[...] + bb0_ref[...], 0.0)
    h = jnp.maximum(h @ wb1_ref[...] + bb1_ref[...], 0.0)
    dout = h @ wb2_ref[...] + bb2_ref[...]

    xs_ref[:, 0, :] = dout
    xs_ref[:, 1:, :] = sparse_ref[...]
    x = xs_ref[...]

    g = lax.dot_general(x, x, (((2,), (2,)), ((0,), (0,))),
                        preferred_element_type=jnp.float32)

    t = dout @ wt0d_ref[...] + bt0_ref[...]
    for i in range(N_CONCAT):
        t += g[:, i, :] @ ahat_ref[i]
    h1 = jnp.maximum(t, 0.0)
    h2 = jnp.maximum(h1 @ wt1_ref[...] + bt1_ref[...], 0.0)
    out_ref[...] = h2 @ wt2_ref[...] + bt2_ref[...]


@jax.jit
def _dense_tail(dense_x, sparse_out, Wb0p, bb0, Wb1, bb1, Wb2, bb2,
                Ahat, Wt0d, bt0, Wt1, bt1, Wt2, bt2):
    nblk = BATCH // BB
    whole = lambda *shape: pl.BlockSpec(shape, lambda i: (0,) * len(shape))
    out = pl.pallas_call(
        _dense_body,
        grid=(nblk,),
        in_specs=[
            pl.BlockSpec((BB, 128), lambda i: (i, 0)),
            pl.BlockSpec((BB, N_FIELDS, EMBED_DIM), lambda i: (i, 0, 0)),
            whole(128, 512), whole(1, 512),
            whole(512, 256), whole(1, 256),
            whole(256, 32), whole(1, 32),
            whole(N_CONCAT, N_CONCAT, 512),
            whole(32, 512), whole(1, 512),
            whole(512, 256), whole(1, 256),
            whole(256, 1), whole(1, 1),
        ],
        out_specs=pl.BlockSpec((BB, 1), lambda i: (i, 0)),
        out_shape=jax.ShapeDtypeStruct((BATCH, 1), jnp.float32),
        scratch_shapes=[pltpu.VMEM((BB, N_CONCAT, EMBED_DIM), jnp.float32)],
    )(dense_x, sparse_out, Wb0p, bb0, Wb1, bb1, Wb2, bb2,
      Ahat, Wt0d, bt0, Wt1, bt1, Wt2, bt2)
    return out[:, 0]


def kernel(dense_x, sparse_x, embedding_table,
           Wb0, bb0, Wb1, bb1, Wb2, bb2,
           Wt0, bt0, Wt1, bt1, Wt2, bt2):
    offsets = jnp.asarray(np.concatenate([[0], np.cumsum(NUM_EMB)[:-1]]),
                          dtype=sparse_x.dtype)
    indices = sparse_x + offsets[None, :]
    sparse_out = jnp.take(embedding_table, indices, axis=0)

    dense_xp = jnp.pad(dense_x, ((0, 0), (0, 128 - dense_x.shape[1])))
    Wb0p = jnp.pad(Wb0, ((0, 128 - Wb0.shape[0]), (0, 0)))

    iu = np.triu_indices(N_CONCAT, k=1)
    Ahat = jnp.zeros((N_CONCAT, N_CONCAT, 512), jnp.float32)
    Ahat = Ahat.at[iu[0], iu[1]].set(Wt0[:351])
    Ahat = 0.5 * (Ahat + jnp.transpose(Ahat, (1, 0, 2)))
    Wt0d = Wt0[351:]

    out = _dense_tail(dense_xp, sparse_out, Wb0p, bb0[None, :], Wb1,
                      bb1[None, :], Wb2, bb2[None, :], Ahat, Wt0d,
                      bt0[None, :], Wt1, bt1[None, :], Wt2, bt2[None, :])
    return out
